# Initial kernel scaffold; baseline (speedup 1.0000x reference)
import jax
import jax.numpy as jnp
from jax.experimental import pallas as pl


def kernel(z_e, embedding_weight):
    raise NotImplementedError("write your pallas kernel here")


# speedup vs baseline: 1.0953x; 1.0953x over previous
import jax
import jax.numpy as jnp
from jax.experimental import pallas as pl
from jax.experimental.pallas import tpu as pltpu

_K = 1024
_D = 64
_B = 16
_H = 32
_W = 32
_N = _B * _H * _W
_T = 256
_STEPS = _N // _T
_COMMIT = 0.25


def _vq_body(z_ref, e_ref, enc_ref, q_ref, loss_ref, perp_ref, cnt_ref, sq_ref):
    s = pl.program_id(0)
    zt = z_ref[...]
    e = e_ref[...]
    p = jax.lax.dot_general(zt, e, (((1,), (1,)), ((), ())),
                            preferred_element_type=jnp.float32)
    sz = jnp.sum(zt * zt, axis=1, keepdims=True)
    se = jnp.sum(e * e, axis=1)
    dist = (sz + se[None, :]) - 2.0 * p
    dmin = jnp.min(dist, axis=1, keepdims=True)
    kio = jax.lax.broadcasted_iota(jnp.int32, (_T, _K), 1)
    idx = jnp.min(jnp.where(dist == dmin, kio, _K), axis=1, keepdims=True)
    onehot = (kio == idx).astype(jnp.float32)
    enc_ref[...] = onehot
    q = jax.lax.dot_general(onehot, e, (((1,), (0,)), ((), ())),
                            preferred_element_type=jnp.float32)
    diff = q - zt
    q_ref[...] = zt + diff
    bc = jnp.sum(onehot, axis=0, keepdims=True)
    bs = jnp.sum(jnp.sum(diff * diff, axis=1, keepdims=True),
                 axis=0, keepdims=True)

    @pl.when(s == 0)
    def _():
        cnt_ref[...] = bc
        sq_ref[...] = bs

    @pl.when(s > 0)
    def _():
        cnt_ref[...] += bc
        sq_ref[...] += bs

    @pl.when(s == _STEPS - 1)
    def _():
        avg = cnt_ref[...] * (1.0 / _N)
        ent = jnp.sum(avg * jnp.log(avg + 1e-10), axis=1, keepdims=True)
        perp_ref[...] = jnp.exp(-ent)
        loss_ref[...] = sq_ref[...] * (_COMMIT / (_N * _D))


def kernel(z_e, embedding_weight):
    z_flat = z_e.transpose(0, 2, 3, 1).reshape(_N, _D)
    enc, qf, loss, perp = pl.pallas_call(
        _vq_body,
        grid=(_STEPS,),
        in_specs=[pl.BlockSpec((_T, _D), lambda s: (s, 0)),
                  pl.BlockSpec((_K, _D), lambda s: (0, 0))],
        out_specs=[pl.BlockSpec((_T, _K), lambda s: (s, 0)),
                   pl.BlockSpec((_T, _D), lambda s: (s, 0)),
                   pl.BlockSpec((1, 1), lambda s: (0, 0)),
                   pl.BlockSpec((1, 1), lambda s: (0, 0))],
        out_shape=[jax.ShapeDtypeStruct((_N, _K), jnp.float32),
                   jax.ShapeDtypeStruct((_N, _D), jnp.float32),
                   jax.ShapeDtypeStruct((1, 1), jnp.float32),
                   jax.ShapeDtypeStruct((1, 1), jnp.float32)],
        scratch_shapes=[pltpu.VMEM((1, _K), jnp.float32),
                        pltpu.VMEM((1, 1), jnp.float32)],
        compiler_params=pltpu.CompilerParams(
            dimension_semantics=("arbitrary",)),
    )(z_flat, embedding_weight)
    q_out = qf.reshape(_B, _H, _W, _D).transpose(0, 3, 1, 2)
    return (q_out, loss[0, 0], perp[0, 0], enc)


# speedup vs baseline: 1.3569x; 1.2389x over previous
import jax
import jax.numpy as jnp
from jax.experimental import pallas as pl
from jax.experimental.pallas import tpu as pltpu

_K = 1024
_D = 64
_B = 16
_H = 32
_W = 32
_N = _B * _H * _W
_T = 512
_STEPS = _N // _T
_COMMIT = 0.25


def _vq_body(z_ref, e_ref, enc_ref, q_ref, loss_ref, perp_ref, cnt_ref, sq_ref):
    s = pl.program_id(0)
    zt = z_ref[...]
    e = e_ref[...]
    p = jax.lax.dot_general(zt, e, (((1,), (1,)), ((), ())),
                            preferred_element_type=jnp.float32)
    sz = jnp.sum(zt * zt, axis=1, keepdims=True)
    se = jnp.sum(e * e, axis=1)
    dist = (sz + se[None, :]) - 2.0 * p
    dmin = jnp.min(dist, axis=1, keepdims=True)
    kio = jax.lax.broadcasted_iota(jnp.int32, (_T, _K), 1)
    idx = jnp.min(jnp.where(dist == dmin, kio, _K), axis=1, keepdims=True)
    onehot = (kio == idx).astype(jnp.float32)
    enc_ref[...] = onehot
    q = jax.lax.dot_general(onehot, e, (((1,), (0,)), ((), ())),
                            preferred_element_type=jnp.float32)
    diff = q - zt
    q_ref[...] = zt + diff
    bc = jnp.sum(onehot, axis=0, keepdims=True)
    bs = jnp.sum(jnp.sum(diff * diff, axis=1, keepdims=True),
                 axis=0, keepdims=True)

    @pl.when(s == 0)
    def _():
        cnt_ref[...] = bc
        sq_ref[...] = bs

    @pl.when(s > 0)
    def _():
        cnt_ref[...] += bc
        sq_ref[...] += bs

    @pl.when(s == _STEPS - 1)
    def _():
        avg = cnt_ref[...] * (1.0 / _N)
        ent = jnp.sum(avg * jnp.log(avg + 1e-10), axis=1, keepdims=True)
        perp_ref[...] = jnp.exp(-ent)
        loss_ref[...] = sq_ref[...] * (_COMMIT / (_N * _D))


def kernel(z_e, embedding_weight):
    z_flat = z_e.transpose(0, 2, 3, 1).reshape(_N, _D)
    enc, qf, loss, perp = pl.pallas_call(
        _vq_body,
        grid=(_STEPS,),
        in_specs=[pl.BlockSpec((_T, _D), lambda s: (s, 0)),
                  pl.BlockSpec((_K, _D), lambda s: (0, 0))],
        out_specs=[pl.BlockSpec((_T, _K), lambda s: (s, 0)),
                   pl.BlockSpec((_T, _D), lambda s: (s, 0)),
                   pl.BlockSpec((1, 1), lambda s: (0, 0)),
                   pl.BlockSpec((1, 1), lambda s: (0, 0))],
        out_shape=[jax.ShapeDtypeStruct((_N, _K), jnp.float32),
                   jax.ShapeDtypeStruct((_N, _D), jnp.float32),
                   jax.ShapeDtypeStruct((1, 1), jnp.float32),
                   jax.ShapeDtypeStruct((1, 1), jnp.float32)],
        scratch_shapes=[pltpu.VMEM((1, _K), jnp.float32),
                        pltpu.VMEM((1, 1), jnp.float32)],
        compiler_params=pltpu.CompilerParams(
            dimension_semantics=("arbitrary",)),
    )(z_flat, embedding_weight)
    q_out = qf.reshape(_B, _H, _W, _D).transpose(0, 3, 1, 2)
    return (q_out, loss[0, 0], perp[0, 0], enc)


# speedup vs baseline: 1.5357x; 1.1318x over previous
import jax
import jax.numpy as jnp
from jax.experimental import pallas as pl
from jax.experimental.pallas import tpu as pltpu

_K = 1024
_D = 64
_B = 16
_H = 32
_W = 32
_N = _B * _H * _W
_T = 1024
_STEPS = _N // _T
_COMMIT = 0.25


def _vq_body(z_ref, e_ref, enc_ref, q_ref, loss_ref, perp_ref, cnt_ref, sq_ref):
    s = pl.program_id(0)
    zt = z_ref[...]
    e = e_ref[...]
    p = jax.lax.dot_general(zt, e, (((1,), (1,)), ((), ())),
                            preferred_element_type=jnp.float32)
    sz = jnp.sum(zt * zt, axis=1, keepdims=True)
    se = jnp.sum(e * e, axis=1)
    dist = (sz + se[None, :]) - 2.0 * p
    dmin = jnp.min(dist, axis=1, keepdims=True)
    kio = jax.lax.broadcasted_iota(jnp.int32, (_T, _K), 1)
    idx = jnp.min(jnp.where(dist == dmin, kio, _K), axis=1, keepdims=True)
    onehot = (kio == idx).astype(jnp.float32)
    enc_ref[...] = onehot
    q = jax.lax.dot_general(onehot, e, (((1,), (0,)), ((), ())),
                            preferred_element_type=jnp.float32)
    diff = q - zt
    q_ref[...] = zt + diff
    bc = jnp.sum(onehot, axis=0, keepdims=True)
    bs = jnp.sum(jnp.sum(diff * diff, axis=1, keepdims=True),
                 axis=0, keepdims=True)

    @pl.when(s == 0)
    def _():
        cnt_ref[...] = bc
        sq_ref[...] = bs

    @pl.when(s > 0)
    def _():
        cnt_ref[...] += bc
        sq_ref[...] += bs

    @pl.when(s == _STEPS - 1)
    def _():
        avg = cnt_ref[...] * (1.0 / _N)
        ent = jnp.sum(avg * jnp.log(avg + 1e-10), axis=1, keepdims=True)
        perp_ref[...] = jnp.exp(-ent)
        loss_ref[...] = sq_ref[...] * (_COMMIT / (_N * _D))


def kernel(z_e, embedding_weight):
    z_flat = z_e.transpose(0, 2, 3, 1).reshape(_N, _D)
    enc, qf, loss, perp = pl.pallas_call(
        _vq_body,
        grid=(_STEPS,),
        in_specs=[pl.BlockSpec((_T, _D), lambda s: (s, 0)),
                  pl.BlockSpec((_K, _D), lambda s: (0, 0))],
        out_specs=[pl.BlockSpec((_T, _K), lambda s: (s, 0)),
                   pl.BlockSpec((_T, _D), lambda s: (s, 0)),
                   pl.BlockSpec((1, 1), lambda s: (0, 0)),
                   pl.BlockSpec((1, 1), lambda s: (0, 0))],
        out_shape=[jax.ShapeDtypeStruct((_N, _K), jnp.float32),
                   jax.ShapeDtypeStruct((_N, _D), jnp.float32),
                   jax.ShapeDtypeStruct((1, 1), jnp.float32),
                   jax.ShapeDtypeStruct((1, 1), jnp.float32)],
        scratch_shapes=[pltpu.VMEM((1, _K), jnp.float32),
                        pltpu.VMEM((1, 1), jnp.float32)],
        compiler_params=pltpu.CompilerParams(
            dimension_semantics=("arbitrary",)),
    )(z_flat, embedding_weight)
    q_out = qf.reshape(_B, _H, _W, _D).transpose(0, 3, 1, 2)
    return (q_out, loss[0, 0], perp[0, 0], enc)


# speedup vs baseline: 1.6204x; 1.0552x over previous
import jax
import jax.numpy as jnp
from jax.experimental import pallas as pl
from jax.experimental.pallas import tpu as pltpu

_K = 1024
_D = 64
_B = 16
_H = 32
_W = 32
_N = _B * _H * _W
_T = 2048
_STEPS = _N // _T
_COMMIT = 0.25


def _vq_body(z_ref, e_ref, enc_ref, q_ref, loss_ref, perp_ref, cnt_ref, sq_ref):
    s = pl.program_id(0)
    zt = z_ref[...]
    e = e_ref[...]
    p = jax.lax.dot_general(zt, e, (((1,), (1,)), ((), ())),
                            preferred_element_type=jnp.float32)
    sz = jnp.sum(zt * zt, axis=1, keepdims=True)
    se = jnp.sum(e * e, axis=1)
    dist = (sz + se[None, :]) - 2.0 * p
    dmin = jnp.min(dist, axis=1, keepdims=True)
    kio = jax.lax.broadcasted_iota(jnp.int32, (_T, _K), 1)
    idx = jnp.min(jnp.where(dist == dmin, kio, _K), axis=1, keepdims=True)
    onehot = (kio == idx).astype(jnp.float32)
    enc_ref[...] = onehot
    q = jax.lax.dot_general(onehot, e, (((1,), (0,)), ((), ())),
                            preferred_element_type=jnp.float32)
    diff = q - zt
    q_ref[...] = zt + diff
    bc = jnp.sum(onehot, axis=0, keepdims=True)
    bs = jnp.sum(jnp.sum(diff * diff, axis=1, keepdims=True),
                 axis=0, keepdims=True)

    @pl.when(s == 0)
    def _():
        cnt_ref[...] = bc
        sq_ref[...] = bs

    @pl.when(s > 0)
    def _():
        cnt_ref[...] += bc
        sq_ref[...] += bs

    @pl.when(s == _STEPS - 1)
    def _():
        avg = cnt_ref[...] * (1.0 / _N)
        ent = jnp.sum(avg * jnp.log(avg + 1e-10), axis=1, keepdims=True)
        perp_ref[...] = jnp.exp(-ent)
        loss_ref[...] = sq_ref[...] * (_COMMIT / (_N * _D))


def kernel(z_e, embedding_weight):
    z_flat = z_e.transpose(0, 2, 3, 1).reshape(_N, _D)
    enc, qf, loss, perp = pl.pallas_call(
        _vq_body,
        grid=(_STEPS,),
        in_specs=[pl.BlockSpec((_T, _D), lambda s: (s, 0)),
                  pl.BlockSpec((_K, _D), lambda s: (0, 0))],
        out_specs=[pl.BlockSpec((_T, _K), lambda s: (s, 0)),
                   pl.BlockSpec((_T, _D), lambda s: (s, 0)),
                   pl.BlockSpec((1, 1), lambda s: (0, 0)),
                   pl.BlockSpec((1, 1), lambda s: (0, 0))],
        out_shape=[jax.ShapeDtypeStruct((_N, _K), jnp.float32),
                   jax.ShapeDtypeStruct((_N, _D), jnp.float32),
                   jax.ShapeDtypeStruct((1, 1), jnp.float32),
                   jax.ShapeDtypeStruct((1, 1), jnp.float32)],
        scratch_shapes=[pltpu.VMEM((1, _K), jnp.float32),
                        pltpu.VMEM((1, 1), jnp.float32)],
        compiler_params=pltpu.CompilerParams(
            dimension_semantics=("arbitrary",)),
    )(z_flat, embedding_weight)
    q_out = qf.reshape(_B, _H, _W, _D).transpose(0, 3, 1, 2)
    return (q_out, loss[0, 0], perp[0, 0], enc)
